# baseline (device time: 87897 ns/iter reference)
import jax
import jax.numpy as jnp
from jax import lax
from jax.experimental import pallas as pl
from jax.experimental.pallas import tpu as pltpu

N_DEV = 4

_CompilerParams = getattr(pltpu, "CompilerParams", None) or getattr(
    pltpu, "TPUCompilerParams"
)


def kernel(x, w_mat, scale_x, scale_w):
    m_per, k = x.shape
    _, n_per = w_mat.shape
    m_half = m_per // 2

    def body(x_ref, w_ref, sx_ref, sw_ref, out_ref,
             cw_ref, ccw_ref, cw_send, cw_recv, ccw_send, ccw_recv):
        my = lax.axis_index("i")
        left = lax.rem(my + (N_DEV - 1), N_DEV)
        right = lax.rem(my + 1, N_DEV)

        barrier = pltpu.get_barrier_semaphore()
        for nbr in (left, right):
            pl.semaphore_signal(
                barrier, inc=1,
                device_id=(nbr,), device_id_type=pl.DeviceIdType.MESH,
            )
        pl.semaphore_wait(barrier, 2)

        scale = sx_ref[0] * sw_ref[0]

        cw_ref[0, :, :] = x_ref[pl.ds(0, m_half), :]
        ccw_ref[0, :, :] = x_ref[pl.ds(m_half, m_half), :]

        def do_half(chunk, origin, is_top):
            acc = lax.dot_general(
                chunk, w_ref[:, :],
                dimension_numbers=(((1,), (0,)), ((), ())),
                preferred_element_type=jnp.int32,
            )
            y = jnp.maximum(acc.astype(jnp.float32) * scale, 0.0)
            base = origin * m_per + (0 if is_top else m_half)
            out_ref[pl.ds(base, m_half), :] = y

        for h in range(N_DEV - 1):
            cw_rdma = pltpu.make_async_remote_copy(
                src_ref=cw_ref.at[h],
                dst_ref=cw_ref.at[h + 1],
                send_sem=cw_send.at[h],
                recv_sem=cw_recv.at[h],
                device_id=(right,),
                device_id_type=pl.DeviceIdType.MESH,
            )
            ccw_rdma = pltpu.make_async_remote_copy(
                src_ref=ccw_ref.at[h],
                dst_ref=ccw_ref.at[h + 1],
                send_sem=ccw_send.at[h],
                recv_sem=ccw_recv.at[h],
                device_id=(left,),
                device_id_type=pl.DeviceIdType.MESH,
            )
            cw_rdma.start()
            ccw_rdma.start()

            do_half(cw_ref[h, :, :], lax.rem(my + (N_DEV - h), N_DEV), True)
            do_half(ccw_ref[h, :, :], lax.rem(my + h, N_DEV), False)

            cw_rdma.wait()
            ccw_rdma.wait()

        last = N_DEV - 1
        do_half(cw_ref[last, :, :], lax.rem(my + 1, N_DEV), True)
        do_half(ccw_ref[last, :, :], lax.rem(my + last, N_DEV), False)

    return pl.pallas_call(
        body,
        out_shape=jax.ShapeDtypeStruct((N_DEV * m_per, n_per), jnp.float32),
        in_specs=[
            pl.BlockSpec(memory_space=pltpu.VMEM),
            pl.BlockSpec(memory_space=pltpu.VMEM),
            pl.BlockSpec(memory_space=pltpu.SMEM),
            pl.BlockSpec(memory_space=pltpu.SMEM),
        ],
        out_specs=pl.BlockSpec(memory_space=pltpu.VMEM),
        scratch_shapes=[
            pltpu.VMEM((N_DEV, m_half, k), jnp.int8),
            pltpu.VMEM((N_DEV, m_half, k), jnp.int8),
            pltpu.SemaphoreType.DMA((N_DEV - 1,)),
            pltpu.SemaphoreType.DMA((N_DEV - 1,)),
            pltpu.SemaphoreType.DMA((N_DEV - 1,)),
            pltpu.SemaphoreType.DMA((N_DEV - 1,)),
        ],
        compiler_params=_CompilerParams(collective_id=0),
    )(x, w_mat, scale_x, scale_w)


# device time: 84365 ns/iter; 1.0419x vs baseline; 1.0419x over previous
import jax
import jax.numpy as jnp
from jax import lax
from jax.experimental import pallas as pl
from jax.experimental.pallas import tpu as pltpu

N_DEV = 4
SUBS = 4

_CompilerParams = getattr(pltpu, "CompilerParams", None) or getattr(
    pltpu, "TPUCompilerParams"
)


def kernel(x, w_mat, scale_x, scale_w):
    m_per, k = x.shape
    _, n_per = w_mat.shape
    m_half = m_per // 2
    m_sub = m_half // SUBS

    def body(x_ref, w_ref, sx_ref, sw_ref, out_ref,
             cw_ref, ccw_ref, cw_send, cw_recv, ccw_send, ccw_recv,
             sub_send_cw, sub_recv_cw, sub_send_ccw, sub_recv_ccw):
        my = lax.axis_index("i")
        left = lax.rem(my + (N_DEV - 1), N_DEV)
        right = lax.rem(my + 1, N_DEV)

        barrier = pltpu.get_barrier_semaphore()
        for nbr in (left, right):
            pl.semaphore_signal(
                barrier, inc=1,
                device_id=(nbr,), device_id_type=pl.DeviceIdType.MESH,
            )
        pl.semaphore_wait(barrier, 2)

        scale = sx_ref[0] * sw_ref[0]

        def do_rows(chunk, origin, row_off, rows):
            acc = lax.dot_general(
                chunk, w_ref[:, :],
                dimension_numbers=(((1,), (0,)), ((), ())),
                preferred_element_type=jnp.int32,
            )
            y = jnp.maximum(acc.astype(jnp.float32) * scale, 0.0)
            out_ref[pl.ds(origin * m_per + row_off, rows), :] = y

        deferred = []

        for h in range(N_DEV - 2):
            if h == 0:
                cw_src = x_ref.at[pl.ds(0, m_half), :]
                ccw_src = x_ref.at[pl.ds(m_half, m_half), :]
            else:
                cw_src = cw_ref.at[h]
                ccw_src = ccw_ref.at[h]
            cw_rdma = pltpu.make_async_remote_copy(
                src_ref=cw_src,
                dst_ref=cw_ref.at[h + 1],
                send_sem=cw_send.at[h],
                recv_sem=cw_recv.at[h],
                device_id=(right,),
                device_id_type=pl.DeviceIdType.MESH,
            )
            ccw_rdma = pltpu.make_async_remote_copy(
                src_ref=ccw_src,
                dst_ref=ccw_ref.at[h + 1],
                send_sem=ccw_send.at[h],
                recv_sem=ccw_recv.at[h],
                device_id=(left,),
                device_id_type=pl.DeviceIdType.MESH,
            )
            cw_rdma.start()
            ccw_rdma.start()
            deferred += [cw_rdma, ccw_rdma]

            if h == 0:
                do_rows(x_ref[pl.ds(0, m_half), :], my, 0, m_half)
                do_rows(x_ref[pl.ds(m_half, m_half), :], my, m_half, m_half)
            else:
                do_rows(cw_ref[h, :, :],
                        lax.rem(my + (N_DEV - h), N_DEV), 0, m_half)
                do_rows(ccw_ref[h, :, :],
                        lax.rem(my + h, N_DEV), m_half, m_half)

            cw_rdma.wait_recv()
            ccw_rdma.wait_recv()

        lh = N_DEV - 2
        subs = []
        for s in range(SUBS):
            rows = pl.ds(s * m_sub, m_sub)
            cw_sub = pltpu.make_async_remote_copy(
                src_ref=cw_ref.at[lh, rows, :],
                dst_ref=cw_ref.at[lh + 1, rows, :],
                send_sem=sub_send_cw.at[s],
                recv_sem=sub_recv_cw.at[s],
                device_id=(right,),
                device_id_type=pl.DeviceIdType.MESH,
            )
            ccw_sub = pltpu.make_async_remote_copy(
                src_ref=ccw_ref.at[lh, rows, :],
                dst_ref=ccw_ref.at[lh + 1, rows, :],
                send_sem=sub_send_ccw.at[s],
                recv_sem=sub_recv_ccw.at[s],
                device_id=(left,),
                device_id_type=pl.DeviceIdType.MESH,
            )
            cw_sub.start()
            ccw_sub.start()
            subs.append((cw_sub, ccw_sub))
        deferred += [d for pair in subs for d in pair]

        diag = lax.rem(my + 2, N_DEV)
        do_rows(cw_ref[lh, :, :], diag, 0, m_half)
        do_rows(ccw_ref[lh, :, :], diag, m_half, m_half)

        cw_last = lax.rem(my + 1, N_DEV)
        ccw_last = lax.rem(my + 3, N_DEV)
        for s in range(SUBS):
            cw_sub, ccw_sub = subs[s]
            cw_sub.wait_recv()
            do_rows(cw_ref[lh + 1, pl.ds(s * m_sub, m_sub), :],
                    cw_last, s * m_sub, m_sub)
            ccw_sub.wait_recv()
            do_rows(ccw_ref[lh + 1, pl.ds(s * m_sub, m_sub), :],
                    ccw_last, m_half + s * m_sub, m_sub)

        for d in deferred:
            d.wait_send()

    return pl.pallas_call(
        body,
        out_shape=jax.ShapeDtypeStruct((N_DEV * m_per, n_per), jnp.float32),
        in_specs=[
            pl.BlockSpec(memory_space=pltpu.VMEM),
            pl.BlockSpec(memory_space=pltpu.VMEM),
            pl.BlockSpec(memory_space=pltpu.SMEM),
            pl.BlockSpec(memory_space=pltpu.SMEM),
        ],
        out_specs=pl.BlockSpec(memory_space=pltpu.VMEM),
        scratch_shapes=[
            pltpu.VMEM((N_DEV, m_half, k), jnp.int8),
            pltpu.VMEM((N_DEV, m_half, k), jnp.int8),
            pltpu.SemaphoreType.DMA((N_DEV - 2,)),
            pltpu.SemaphoreType.DMA((N_DEV - 2,)),
            pltpu.SemaphoreType.DMA((N_DEV - 2,)),
            pltpu.SemaphoreType.DMA((N_DEV - 2,)),
            pltpu.SemaphoreType.DMA((SUBS,)),
            pltpu.SemaphoreType.DMA((SUBS,)),
            pltpu.SemaphoreType.DMA((SUBS,)),
            pltpu.SemaphoreType.DMA((SUBS,)),
        ],
        compiler_params=_CompilerParams(collective_id=0),
    )(x, w_mat, scale_x, scale_w)
